# Initial kernel scaffold; baseline (speedup 1.0000x reference)
import jax
import jax.numpy as jnp
from jax.experimental import pallas as pl


def kernel(rel_embs, ent_embs, r_to_e_flat, seg_ids, e_r_bias, num_rels, W_ih, W_hh, b_ih, b_hh):
    raise NotImplementedError("write your pallas kernel here")


# speedup vs baseline: 2.2232x; 2.2232x over previous
import functools

import jax
import jax.numpy as jnp
from jax import lax
from jax.experimental import pallas as pl
from jax.experimental.pallas import tpu as pltpu
from jax.experimental.pallas import tpu_sc as plsc

R = 500
H = 256
N = 10000
E = 160000
R_PAD = 512

NC = 2
NS = 16
NW = NC * NS
K = 128
NB = -(-E // (NW * K))
E_PAD = NW * NB * K
B = 512
NBLK = E_PAD // B


def _make_sc_gather():
    mesh = plsc.VectorSubcoreMesh(core_axis_name="c", subcore_axis_name="s")

    @functools.partial(
        pl.kernel,
        mesh=mesh,
        out_type=jax.ShapeDtypeStruct((NW, NB, K, H), jnp.float32),
        scratch_types=[
            pltpu.VMEM((K,), jnp.int32),
            pltpu.VMEM((K, H), jnp.float32),
            pltpu.SemaphoreType.DMA,
        ],
    )
    def gatherk(ent_hbm, idx_hbm, rows_out, idx_blk, rows_v, sem):
        c = lax.axis_index("c")
        s = lax.axis_index("s")
        wid = s * NC + c

        def body(j, carry):
            pltpu.sync_copy(idx_hbm.at[wid, j], idx_blk)
            pltpu.async_copy(ent_hbm.at[idx_blk], rows_v, sem).wait()
            pltpu.sync_copy(rows_v, rows_out.at[wid, j])
            return carry

        lax.fori_loop(0, NB, body, 0)

    return gatherk


_GATHER = _make_sc_gather()


def _segsum_gru_body(seg_ref, gath_ref, rel_ref, wih_ref, whh_ref, bih_ref,
                     bhh_ref, out_ref, acc, cnt):
    i = pl.program_id(0)

    @pl.when(i == 0)
    def _init():
        acc[...] = jnp.zeros((R_PAD, H), jnp.float32)
        cnt[...] = jnp.zeros((R_PAD, 1), jnp.float32)

    seg = seg_ref[0]
    m = lax.broadcasted_iota(jnp.int32, (R_PAD, B), 0) == seg
    ohf = jnp.where(m, 1.0, 0.0)
    rows = gath_ref[0]
    dn = (((1,), (0,)), ((), ()))
    acc[...] += lax.dot_general(ohf.astype(jnp.bfloat16),
                                rows.astype(jnp.bfloat16), dn,
                                preferred_element_type=jnp.float32)
    cnt[...] += jnp.sum(ohf, axis=1, keepdims=True)

    @pl.when(i == NBLK - 1)
    def _epilogue():
        sums = acc[...]
        mean = sums / jnp.maximum(cnt[...], 1.0)
        rel = rel_ref[...]
        wih = wih_ref[...]
        whh = whh_ref[...]
        dnt = (((1,), (1,)), ((), ()))
        gi = lax.dot_general(rel, wih[:, :H], dnt,
                             preferred_element_type=jnp.float32)
        gi = gi + lax.dot_general(mean, wih[:, H:], dnt,
                                  preferred_element_type=jnp.float32)
        gi = gi + bih_ref[...]
        gh = lax.dot_general(rel, whh, dnt, preferred_element_type=jnp.float32)
        gh = gh + bhh_ref[...]
        r = jax.nn.sigmoid(gi[:, :H] + gh[:, :H])
        z = jax.nn.sigmoid(gi[:, H:2 * H] + gh[:, H:2 * H])
        n = jnp.tanh(gi[:, 2 * H:] + r * gh[:, 2 * H:])
        h0 = (1.0 - z) * n + z * rel
        norm = jnp.sqrt(jnp.sum(h0 * h0, axis=1, keepdims=True))
        out_ref[...] = h0 / jnp.maximum(norm, 1e-12)


_SEGGRU = pl.pallas_call(
    _segsum_gru_body,
    grid=(NBLK,),
    in_specs=[
        pl.BlockSpec((1, 1, B), lambda i: (i, 0, 0)),
        pl.BlockSpec((1, B, H), lambda i: (i, 0, 0)),
        pl.BlockSpec((R_PAD, H), lambda i: (0, 0)),
        pl.BlockSpec((3 * H, 2 * H), lambda i: (0, 0)),
        pl.BlockSpec((3 * H, H), lambda i: (0, 0)),
        pl.BlockSpec((1, 3 * H), lambda i: (0, 0)),
        pl.BlockSpec((1, 3 * H), lambda i: (0, 0)),
    ],
    out_specs=pl.BlockSpec((R_PAD, H), lambda i: (0, 0)),
    out_shape=jax.ShapeDtypeStruct((R_PAD, H), jnp.float32),
    scratch_shapes=[
        pltpu.VMEM((R_PAD, H), jnp.float32),
        pltpu.VMEM((R_PAD, 1), jnp.float32),
    ],
)


def kernel(rel_embs, ent_embs, r_to_e_flat, seg_ids, e_r_bias, num_rels,
           W_ih, W_hh, b_ih, b_hh):
    pad = E_PAD - E
    idx_rs = jnp.concatenate(
        [r_to_e_flat.astype(jnp.int32),
         jnp.zeros((pad,), jnp.int32)]).reshape(NW, NB, K)
    seg_rs = jnp.concatenate(
        [seg_ids.astype(jnp.int32),
         jnp.full((pad,), R_PAD - 1, jnp.int32)]).reshape(NBLK, 1, B)
    gathered = _GATHER(ent_embs, idx_rs)
    rel_pad = jnp.pad(rel_embs, ((0, R_PAD - R), (0, 0)))
    out = _SEGGRU(seg_rs, gathered.reshape(NBLK, B, H), rel_pad, W_ih, W_hh,
                  b_ih.reshape(1, 3 * H), b_hh.reshape(1, 3 * H))
    return out[:R]


# speedup vs baseline: 2.3547x; 1.0591x over previous
import functools

import jax
import jax.numpy as jnp
from jax import lax
from jax.experimental import pallas as pl
from jax.experimental.pallas import tpu as pltpu
from jax.experimental.pallas import tpu_sc as plsc

R = 500
H = 256
N = 10000
E = 160000
R_PAD = 512

NC = 2
NS = 16
NW = NC * NS
K = 128
NB = -(-E // (NW * K))
E_PAD = NW * NB * K
B = 512
NBLK = E_PAD // B


def _make_sc_gather():
    mesh = plsc.VectorSubcoreMesh(core_axis_name="c", subcore_axis_name="s")

    @functools.partial(
        pl.kernel,
        mesh=mesh,
        out_type=jax.ShapeDtypeStruct((NW, NB, K, H), jnp.float32),
        scratch_types=[
            pltpu.VMEM((NB, K), jnp.int32),
            pltpu.VMEM((K, H), jnp.float32),
            pltpu.VMEM((K, H), jnp.float32),
            pltpu.SemaphoreType.DMA,
            pltpu.SemaphoreType.DMA,
            pltpu.SemaphoreType.DMA,
        ],
    )
    def gatherk(ent_hbm, idx_hbm, rows_out, idx_all, rows0, rows1,
                gsem, wsem0, wsem1):
        c = lax.axis_index("c")
        s = lax.axis_index("s")
        wid = s * NC + c

        pltpu.sync_copy(idx_hbm.at[wid], idx_all)

        rows = (rows0, rows1)
        wsem = (wsem0, wsem1)
        writes = [None, None]
        for j in range(NB):
            b = j & 1
            if writes[b] is not None:
                writes[b].wait()
            pltpu.async_copy(ent_hbm.at[idx_all.at[j]], rows[b], gsem).wait()
            writes[b] = pltpu.async_copy(rows[b], rows_out.at[wid, j],
                                         wsem[b])
        writes[0].wait()
        writes[1].wait()

    return gatherk


_GATHER = _make_sc_gather()


def _segsum_gru_body(seg_ref, gath_ref, rel_ref, wih_ref, whh_ref, bih_ref,
                     bhh_ref, out_ref, acc, cnt):
    i = pl.program_id(0)

    @pl.when(i == 0)
    def _init():
        acc[...] = jnp.zeros((R_PAD, H), jnp.float32)
        cnt[...] = jnp.zeros((R_PAD, 1), jnp.float32)

    seg = seg_ref[0]
    m = lax.broadcasted_iota(jnp.int32, (R_PAD, B), 0) == seg
    ohf = jnp.where(m, 1.0, 0.0)
    rows = gath_ref[0]
    dn = (((1,), (0,)), ((), ()))
    acc[...] += lax.dot_general(ohf.astype(jnp.bfloat16),
                                rows.astype(jnp.bfloat16), dn,
                                preferred_element_type=jnp.float32)
    cnt[...] += jnp.sum(ohf, axis=1, keepdims=True)

    @pl.when(i == NBLK - 1)
    def _epilogue():
        sums = acc[...]
        mean = sums / jnp.maximum(cnt[...], 1.0)
        rel = rel_ref[...]
        wih = wih_ref[...]
        whh = whh_ref[...]
        dnt = (((1,), (1,)), ((), ()))
        gi = lax.dot_general(rel, wih[:, :H], dnt,
                             preferred_element_type=jnp.float32)
        gi = gi + lax.dot_general(mean, wih[:, H:], dnt,
                                  preferred_element_type=jnp.float32)
        gi = gi + bih_ref[...]
        gh = lax.dot_general(rel, whh, dnt, preferred_element_type=jnp.float32)
        gh = gh + bhh_ref[...]
        r = jax.nn.sigmoid(gi[:, :H] + gh[:, :H])
        z = jax.nn.sigmoid(gi[:, H:2 * H] + gh[:, H:2 * H])
        n = jnp.tanh(gi[:, 2 * H:] + r * gh[:, 2 * H:])
        h0 = (1.0 - z) * n + z * rel
        norm = jnp.sqrt(jnp.sum(h0 * h0, axis=1, keepdims=True))
        out_ref[...] = h0 / jnp.maximum(norm, 1e-12)


_SEGGRU = pl.pallas_call(
    _segsum_gru_body,
    grid=(NBLK,),
    in_specs=[
        pl.BlockSpec((1, 1, B), lambda i: (i, 0, 0)),
        pl.BlockSpec((1, B, H), lambda i: (i, 0, 0)),
        pl.BlockSpec((R_PAD, H), lambda i: (0, 0)),
        pl.BlockSpec((3 * H, 2 * H), lambda i: (0, 0)),
        pl.BlockSpec((3 * H, H), lambda i: (0, 0)),
        pl.BlockSpec((1, 3 * H), lambda i: (0, 0)),
        pl.BlockSpec((1, 3 * H), lambda i: (0, 0)),
    ],
    out_specs=pl.BlockSpec((R_PAD, H), lambda i: (0, 0)),
    out_shape=jax.ShapeDtypeStruct((R_PAD, H), jnp.float32),
    scratch_shapes=[
        pltpu.VMEM((R_PAD, H), jnp.float32),
        pltpu.VMEM((R_PAD, 1), jnp.float32),
    ],
)


def kernel(rel_embs, ent_embs, r_to_e_flat, seg_ids, e_r_bias, num_rels,
           W_ih, W_hh, b_ih, b_hh):
    pad = E_PAD - E
    idx_rs = jnp.concatenate(
        [r_to_e_flat.astype(jnp.int32),
         jnp.zeros((pad,), jnp.int32)]).reshape(NW, NB, K)
    seg_rs = jnp.concatenate(
        [seg_ids.astype(jnp.int32),
         jnp.full((pad,), R_PAD - 1, jnp.int32)]).reshape(NBLK, 1, B)
    gathered = _GATHER(ent_embs, idx_rs)
    rel_pad = jnp.pad(rel_embs, ((0, R_PAD - R), (0, 0)))
    out = _SEGGRU(seg_rs, gathered.reshape(NBLK, B, H), rel_pad, W_ih, W_hh,
                  b_ih.reshape(1, 3 * H), b_hh.reshape(1, 3 * H))
    return out[:R]


# speedup vs baseline: 2.4546x; 1.0424x over previous
import functools

import jax
import jax.numpy as jnp
from jax import lax
from jax.experimental import pallas as pl
from jax.experimental.pallas import tpu as pltpu
from jax.experimental.pallas import tpu_sc as plsc

R = 500
H = 256
N = 10000
E = 160000
R_PAD = 512

NC = 2
NS = 16
NW = NC * NS
K = 128
NB = -(-E // (NW * K))
E_PAD = NW * NB * K
B = 512
NBLK = E_PAD // B


def _make_sc_gather():
    mesh = plsc.VectorSubcoreMesh(core_axis_name="c", subcore_axis_name="s")

    @functools.partial(
        pl.kernel,
        mesh=mesh,
        out_type=jax.ShapeDtypeStruct((NW, NB, K, H), jnp.float32),
        scratch_types=[
            pltpu.VMEM((NB, K), jnp.int32),
            pltpu.VMEM((K, H), jnp.float32),
            pltpu.VMEM((K, H), jnp.float32),
            pltpu.VMEM((K, H), jnp.float32),
            pltpu.SemaphoreType.DMA,
            pltpu.SemaphoreType.DMA,
            pltpu.SemaphoreType.DMA,
            pltpu.SemaphoreType.DMA,
            pltpu.SemaphoreType.DMA,
            pltpu.SemaphoreType.DMA,
        ],
    )
    def gatherk(ent_hbm, idx_hbm, rows_out, idx_all, rows0, rows1, rows2,
                gsem0, gsem1, gsem2, wsem0, wsem1, wsem2):
        c = lax.axis_index("c")
        s = lax.axis_index("s")
        wid = s * NC + c

        pltpu.sync_copy(idx_hbm.at[wid], idx_all)

        ND = 3
        rows = (rows0, rows1, rows2)
        gsem = (gsem0, gsem1, gsem2)
        wsem = (wsem0, wsem1, wsem2)
        writes = [None] * ND
        gets = [None] * ND
        for j in range(NB + ND - 1):
            if j < NB:
                b = j % ND
                if writes[b] is not None:
                    writes[b].wait()
                gets[b] = pltpu.async_copy(ent_hbm.at[idx_all.at[j]],
                                           rows[b], gsem[b])
            if j >= ND - 1:
                jd = j - ND + 1
                pb = jd % ND
                gets[pb].wait()
                writes[pb] = pltpu.async_copy(rows[pb],
                                              rows_out.at[wid, jd], wsem[pb])
        for b in range(ND):
            writes[b].wait()

    return gatherk


_GATHER = _make_sc_gather()


def _segsum_gru_body(seg_ref, gath_ref, rel_ref, wih_ref, whh_ref, bih_ref,
                     bhh_ref, out_ref, acc, cnt):
    i = pl.program_id(0)

    @pl.when(i == 0)
    def _init():
        acc[...] = jnp.zeros((R_PAD, H), jnp.float32)
        cnt[...] = jnp.zeros((R_PAD, 1), jnp.float32)

    seg = seg_ref[0]
    m = lax.broadcasted_iota(jnp.int32, (R_PAD, B), 0) == seg
    ohf = jnp.where(m, 1.0, 0.0)
    rows = gath_ref[0]
    dn = (((1,), (0,)), ((), ()))
    acc[...] += lax.dot_general(ohf.astype(jnp.bfloat16),
                                rows.astype(jnp.bfloat16), dn,
                                preferred_element_type=jnp.float32)
    cnt[...] += jnp.sum(ohf, axis=1, keepdims=True)

    @pl.when(i == NBLK - 1)
    def _epilogue():
        sums = acc[...]
        mean = sums / jnp.maximum(cnt[...], 1.0)
        rel = rel_ref[...]
        wih = wih_ref[...]
        whh = whh_ref[...]
        dnt = (((1,), (1,)), ((), ()))
        gi = lax.dot_general(rel, wih[:, :H], dnt,
                             preferred_element_type=jnp.float32)
        gi = gi + lax.dot_general(mean, wih[:, H:], dnt,
                                  preferred_element_type=jnp.float32)
        gi = gi + bih_ref[...]
        gh = lax.dot_general(rel, whh, dnt, preferred_element_type=jnp.float32)
        gh = gh + bhh_ref[...]
        r = jax.nn.sigmoid(gi[:, :H] + gh[:, :H])
        z = jax.nn.sigmoid(gi[:, H:2 * H] + gh[:, H:2 * H])
        n = jnp.tanh(gi[:, 2 * H:] + r * gh[:, 2 * H:])
        h0 = (1.0 - z) * n + z * rel
        norm = jnp.sqrt(jnp.sum(h0 * h0, axis=1, keepdims=True))
        out_ref[...] = h0 / jnp.maximum(norm, 1e-12)


_SEGGRU = pl.pallas_call(
    _segsum_gru_body,
    grid=(NBLK,),
    in_specs=[
        pl.BlockSpec((1, 1, B), lambda i: (i, 0, 0)),
        pl.BlockSpec((1, B, H), lambda i: (i, 0, 0)),
        pl.BlockSpec((R_PAD, H), lambda i: (0, 0)),
        pl.BlockSpec((3 * H, 2 * H), lambda i: (0, 0)),
        pl.BlockSpec((3 * H, H), lambda i: (0, 0)),
        pl.BlockSpec((1, 3 * H), lambda i: (0, 0)),
        pl.BlockSpec((1, 3 * H), lambda i: (0, 0)),
    ],
    out_specs=pl.BlockSpec((R_PAD, H), lambda i: (0, 0)),
    out_shape=jax.ShapeDtypeStruct((R_PAD, H), jnp.float32),
    scratch_shapes=[
        pltpu.VMEM((R_PAD, H), jnp.float32),
        pltpu.VMEM((R_PAD, 1), jnp.float32),
    ],
)


def kernel(rel_embs, ent_embs, r_to_e_flat, seg_ids, e_r_bias, num_rels,
           W_ih, W_hh, b_ih, b_hh):
    pad = E_PAD - E
    idx_rs = jnp.concatenate(
        [r_to_e_flat.astype(jnp.int32),
         jnp.zeros((pad,), jnp.int32)]).reshape(NW, NB, K)
    seg_rs = jnp.concatenate(
        [seg_ids.astype(jnp.int32),
         jnp.full((pad,), R_PAD - 1, jnp.int32)]).reshape(NBLK, 1, B)
    gathered = _GATHER(ent_embs, idx_rs)
    rel_pad = jnp.pad(rel_embs, ((0, R_PAD - R), (0, 0)))
    out = _SEGGRU(seg_rs, gathered.reshape(NBLK, B, H), rel_pad, W_ih, W_hh,
                  b_ih.reshape(1, 3 * H), b_hh.reshape(1, 3 * H))
    return out[:R]


# speedup vs baseline: 2.9274x; 1.1926x over previous
import functools

import jax
import jax.numpy as jnp
from jax import lax
from jax.experimental import pallas as pl
from jax.experimental.pallas import tpu as pltpu
from jax.experimental.pallas import tpu_sc as plsc

R = 500
H = 256
N = 10000
E = 160000
R_PAD = 512

NC = 2
NS = 16
NW = NC * NS
K = 128
NB = -(-E // (NW * K))
E_PAD = NW * NB * K
NBH = NB // 2
B = 512
NBLKH = NW * NBH * K // B


def _make_sc_gather():
    mesh = plsc.VectorSubcoreMesh(core_axis_name="c", subcore_axis_name="s")

    @functools.partial(
        pl.kernel,
        mesh=mesh,
        out_type=jax.ShapeDtypeStruct((NW, NBH, K, H), jnp.float32),
        scratch_types=[
            pltpu.VMEM((NBH, K), jnp.int32),
            pltpu.VMEM((K, H), jnp.float32),
            pltpu.VMEM((K, H), jnp.float32),
            pltpu.VMEM((K, H), jnp.float32),
            pltpu.SemaphoreType.DMA,
            pltpu.SemaphoreType.DMA,
            pltpu.SemaphoreType.DMA,
            pltpu.SemaphoreType.DMA,
            pltpu.SemaphoreType.DMA,
            pltpu.SemaphoreType.DMA,
        ],
    )
    def gatherk(ent_hbm, idx_hbm, rows_out, idx_all, rows0, rows1, rows2,
                gsem0, gsem1, gsem2, wsem0, wsem1, wsem2):
        c = lax.axis_index("c")
        s = lax.axis_index("s")
        wid = s * NC + c

        pltpu.sync_copy(idx_hbm.at[wid], idx_all)

        ND = 3
        rows = (rows0, rows1, rows2)
        gsem = (gsem0, gsem1, gsem2)
        wsem = (wsem0, wsem1, wsem2)
        writes = [None] * ND
        gets = [None] * ND
        for j in range(NBH + ND - 1):
            if j < NBH:
                b = j % ND
                if writes[b] is not None:
                    writes[b].wait()
                gets[b] = pltpu.async_copy(ent_hbm.at[idx_all.at[j]],
                                           rows[b], gsem[b])
            if j >= ND - 1:
                jd = j - ND + 1
                pb = jd % ND
                gets[pb].wait()
                writes[pb] = pltpu.async_copy(rows[pb],
                                              rows_out.at[wid, jd], wsem[pb])
        for b in range(ND):
            writes[b].wait()

    return gatherk


_GATHER = _make_sc_gather()


def _accum_block(seg_ref, gath_ref, i, acc, cnt):
    seg = seg_ref[0]
    m = lax.broadcasted_iota(jnp.int32, (R_PAD, B), 0) == seg
    ohb = jnp.where(m, 1.0, 0.0).astype(jnp.bfloat16)
    rows = gath_ref[0]
    dn = (((1,), (0,)), ((), ()))
    acc[...] += lax.dot_general(ohb, rows.astype(jnp.bfloat16), dn,
                                preferred_element_type=jnp.float32)
    cnt[...] += lax.dot_general(ohb, jnp.ones((B, 128), jnp.bfloat16), dn,
                                preferred_element_type=jnp.float32)[:, :1]


def _segsum1_body(seg_ref, gath_ref, sums_ref, cnt_ref, acc, cnt):
    i = pl.program_id(0)

    @pl.when(i == 0)
    def _init():
        acc[...] = jnp.zeros((R_PAD, H), jnp.float32)
        cnt[...] = jnp.zeros((R_PAD, 1), jnp.float32)

    _accum_block(seg_ref, gath_ref, i, acc, cnt)

    @pl.when(i == NBLKH - 1)
    def _fin():
        sums_ref[...] = acc[...]
        cnt_ref[...] = cnt[...]


def _segsum2_body(seg_ref, gath_ref, sums0_ref, cnt0_ref, rel_ref, wih_ref,
                  whh_ref, bih_ref, bhh_ref, out_ref, acc, cnt):
    i = pl.program_id(0)

    @pl.when(i == 0)
    def _init():
        acc[...] = sums0_ref[...]
        cnt[...] = cnt0_ref[...]

    _accum_block(seg_ref, gath_ref, i, acc, cnt)

    @pl.when(i == NBLKH - 1)
    def _epilogue():
        sums = acc[...]
        mean = sums / jnp.maximum(cnt[...], 1.0)
        rel = rel_ref[...]
        wih = wih_ref[...]
        whh = whh_ref[...]
        dnt = (((1,), (1,)), ((), ()))
        gi = lax.dot_general(rel, wih[:, :H], dnt,
                             preferred_element_type=jnp.float32)
        gi = gi + lax.dot_general(mean, wih[:, H:], dnt,
                                  preferred_element_type=jnp.float32)
        gi = gi + bih_ref[...]
        gh = lax.dot_general(rel, whh, dnt, preferred_element_type=jnp.float32)
        gh = gh + bhh_ref[...]
        r = jax.nn.sigmoid(gi[:, :H] + gh[:, :H])
        z = jax.nn.sigmoid(gi[:, H:2 * H] + gh[:, H:2 * H])
        n = jnp.tanh(gi[:, 2 * H:] + r * gh[:, 2 * H:])
        h0 = (1.0 - z) * n + z * rel
        norm = jnp.sqrt(jnp.sum(h0 * h0, axis=1, keepdims=True))
        out_ref[...] = h0 / jnp.maximum(norm, 1e-12)


_FULL = lambda i: (0, 0)

_SEG1 = pl.pallas_call(
    _segsum1_body,
    grid=(NBLKH,),
    in_specs=[
        pl.BlockSpec((1, 1, B), lambda i: (i, 0, 0)),
        pl.BlockSpec((1, B, H), lambda i: (i, 0, 0)),
    ],
    out_specs=[
        pl.BlockSpec((R_PAD, H), _FULL),
        pl.BlockSpec((R_PAD, 1), _FULL),
    ],
    out_shape=[
        jax.ShapeDtypeStruct((R_PAD, H), jnp.float32),
        jax.ShapeDtypeStruct((R_PAD, 1), jnp.float32),
    ],
    scratch_shapes=[
        pltpu.VMEM((R_PAD, H), jnp.float32),
        pltpu.VMEM((R_PAD, 1), jnp.float32),
    ],
)

_SEG2 = pl.pallas_call(
    _segsum2_body,
    grid=(NBLKH,),
    in_specs=[
        pl.BlockSpec((1, 1, B), lambda i: (i, 0, 0)),
        pl.BlockSpec((1, B, H), lambda i: (i, 0, 0)),
        pl.BlockSpec((R_PAD, H), _FULL),
        pl.BlockSpec((R_PAD, 1), _FULL),
        pl.BlockSpec((R_PAD, H), _FULL),
        pl.BlockSpec((3 * H, 2 * H), _FULL),
        pl.BlockSpec((3 * H, H), _FULL),
        pl.BlockSpec((1, 3 * H), _FULL),
        pl.BlockSpec((1, 3 * H), _FULL),
    ],
    out_specs=pl.BlockSpec((R_PAD, H), _FULL),
    out_shape=jax.ShapeDtypeStruct((R_PAD, H), jnp.float32),
    scratch_shapes=[
        pltpu.VMEM((R_PAD, H), jnp.float32),
        pltpu.VMEM((R_PAD, 1), jnp.float32),
    ],
)


def kernel(rel_embs, ent_embs, r_to_e_flat, seg_ids, e_r_bias, num_rels,
           W_ih, W_hh, b_ih, b_hh):
    pad = E_PAD - E
    idx_rs = jnp.concatenate(
        [r_to_e_flat.astype(jnp.int32),
         jnp.zeros((pad,), jnp.int32)]).reshape(NW, NB, K)
    seg_rs = jnp.concatenate(
        [seg_ids.astype(jnp.int32),
         jnp.full((pad,), R_PAD - 1, jnp.int32)]).reshape(NW, NB, K)
    idx1, idx2 = idx_rs[:, :NBH], idx_rs[:, NBH:]
    seg1 = seg_rs[:, :NBH].reshape(NBLKH, 1, B)
    seg2 = seg_rs[:, NBH:].reshape(NBLKH, 1, B)
    g1 = _GATHER(ent_embs, idx1)
    g2 = _GATHER(ent_embs, idx2)
    sums0, cnt0 = _SEG1(seg1, g1.reshape(NBLKH, B, H))
    rel_pad = jnp.pad(rel_embs, ((0, R_PAD - R), (0, 0)))
    out = _SEG2(seg2, g2.reshape(NBLKH, B, H), sums0, cnt0, rel_pad,
                W_ih, W_hh, b_ih.reshape(1, 3 * H), b_hh.reshape(1, 3 * H))
    return out[:R]
